# baseline (speedup 1.0000x reference)
import functools

import jax
import jax.numpy as jnp
from jax import lax
from jax.experimental import pallas as pl
from jax.experimental.pallas import tpu as pltpu
from jax.experimental.pallas import tpu_sc as plsc

N = 50000
DV = 256
DG = 256
S = 128

NC = 2
NS = 16
NW = NC * NS
SEG_PER_W = S // NW
LANES = 16
NJ = DV // LANES
GROUPS = 4
JPG = NJ // GROUPS
CH = 64
OFF_PAD = 144


def _sc_segment_stats(fv, offsets):
    mesh = plsc.VectorSubcoreMesh(core_axis_name="c", subcore_axis_name="s")
    out_type = tuple(
        jax.ShapeDtypeStruct((NW, 8, DV), jnp.float32) for _ in range(4))

    @functools.partial(
        pl.kernel,
        mesh=mesh,
        out_type=out_type,
        scratch_types=[
            pltpu.VMEM((OFF_PAD,), jnp.int32),
            pltpu.VMEM((CH, DV), jnp.float32),
            pltpu.VMEM((CH, DV), jnp.float32),
            pltpu.VMEM((8, DV), jnp.float32),
            pltpu.VMEM((8, DV), jnp.float32),
            pltpu.VMEM((8, DV), jnp.float32),
            pltpu.VMEM((8, DV), jnp.float32),
            pltpu.SemaphoreType.DMA,
            pltpu.SemaphoreType.DMA,
        ],
    )
    def k(fv_hbm, off_hbm, sum_hbm, cnt_hbm, mn_hbm, mx_hbm,
          off_v, buf0, buf1, s_v, c_v, mn_v, mx_v, sem0, sem1):
        wid = lax.axis_index("s") * NC + lax.axis_index("c")
        pltpu.sync_copy(off_hbm, off_v)

        for kk in range(SEG_PER_W):
            seg = wid * SEG_PER_W + kk
            offv = off_v[pl.ds(seg, LANES)]
            a = offv[0]
            b = offv[1]
            n = b - a
            a8 = (a // 8) * 8
            nch = (b - a8 + CH - 1) // CH
            npair = (nch + 1) // 2

            def issue(c, buf, sem, a8=a8):
                row0 = jnp.minimum(a8 + c * CH, N - CH)
                pltpu.async_copy(fv_hbm.at[pl.ds(row0, CH)], buf, sem)

            def wait(buf, sem):
                pltpu.make_async_copy(
                    fv_hbm.at[pl.ds(0, CH)], buf, sem).wait()

            def process(buf, c, cy, a=a, b=b, a8=a8):
                row0 = a8 + c * CH
                w0 = jnp.minimum(row0, N - CH)
                lo = jnp.maximum(a, row0) - w0
                hi = jnp.maximum(jnp.minimum(b, row0 + CH) - w0, lo)
                cy = list(cy)
                for g in range(GROUPS):
                    sub = tuple(cy[3 * JPG * g: 3 * JPG * (g + 1)])

                    def row_body(r, sc, g=g, buf=buf):
                        out = []
                        for jj in range(JPG):
                            j = JPG * g + jj
                            v = buf[r, pl.ds(LANES * j, LANES)]
                            out += [
                                sc[3 * jj] + v,
                                jnp.minimum(sc[3 * jj + 1], v),
                                jnp.maximum(sc[3 * jj + 2], v),
                            ]
                        return tuple(out)

                    sub = plsc.parallel_loop(
                        lo, hi, 1, unroll=4, carry=sub)(row_body)
                    cy[3 * JPG * g: 3 * JPG * (g + 1)] = list(sub)
                return tuple(cy)

            carry = []
            for _ in range(NJ):
                carry += [
                    jnp.zeros((LANES,), jnp.float32),
                    jnp.full((LANES,), jnp.inf, jnp.float32),
                    jnp.full((LANES,), -jnp.inf, jnp.float32),
                ]
            carry = tuple(carry)

            @pl.when(nch > 0)
            def _():
                issue(0, buf0, sem0)

            def pair_body(p, cy):
                c0 = 2 * p
                wait(buf0, sem0)

                @pl.when(c0 + 1 < nch)
                def _():
                    issue(c0 + 1, buf1, sem1)

                cy = process(buf0, c0, cy)
                c1 = c0 + 1

                @pl.when(c1 < nch)
                def _():
                    wait(buf1, sem1)

                    @pl.when(c1 + 1 < nch)
                    def _():
                        issue(c1 + 1, buf0, sem0)

                cy = process(buf1, c1, cy)
                return cy

            carry = lax.fori_loop(0, npair, pair_body, carry)

            cntf = n.astype(jnp.float32)
            for j in range(NJ):
                ds = pl.ds(LANES * j, LANES)
                s_v[kk, ds] = carry[3 * j]
                mn_v[kk, ds] = carry[3 * j + 1]
                mx_v[kk, ds] = carry[3 * j + 2]
                c_v[kk, ds] = jnp.broadcast_to(cntf, (LANES,))

        pltpu.sync_copy(s_v, sum_hbm.at[wid])
        pltpu.sync_copy(c_v, cnt_hbm.at[wid])
        pltpu.sync_copy(mn_v, mn_hbm.at[wid])
        pltpu.sync_copy(mx_v, mx_hbm.at[wid])

    outs = k(fv, offsets)
    return tuple(o[:, :SEG_PER_W].reshape(S, DV) for o in outs)


def _tc_combine(ssum, cnt, mn, mx, W1, W2, W3, bsum):
    def body(s_ref, c_ref, mn_ref, mx_ref, w1_ref, w2_ref, w3_ref, b_ref,
             o_ref):
        c = c_ref[...]
        mean = s_ref[...] / jnp.maximum(c, 1.0)
        ne = c > 0.0
        mnv = jnp.where(ne, mn_ref[...], 0.0)
        mxv = jnp.where(ne, mx_ref[...], 0.0)
        dn = (((1,), (1,)), ((), ()))
        acc = lax.dot_general(mean, w1_ref[...], dn,
                              precision=lax.Precision.HIGHEST,
                              preferred_element_type=jnp.float32)
        acc = acc + lax.dot_general(mnv, w2_ref[...], dn,
                                    precision=lax.Precision.HIGHEST,
                                    preferred_element_type=jnp.float32)
        acc = acc + lax.dot_general(mxv, w3_ref[...], dn,
                                    precision=lax.Precision.HIGHEST,
                                    preferred_element_type=jnp.float32)
        o_ref[...] = acc + b_ref[...]

    return pl.pallas_call(
        body,
        out_shape=jax.ShapeDtypeStruct((S, DG), jnp.float32),
    )(ssum, cnt, mn, mx, W1, W2, W3, bsum)


def kernel(fv, segment_ids, W1, b1, W2, b2, W3, b3):
    ids = segment_ids.astype(jnp.int32)
    off = jnp.searchsorted(ids, jnp.arange(S + 1, dtype=jnp.int32),
                           side="left").astype(jnp.int32)
    off = jnp.concatenate([off, jnp.zeros((OFF_PAD - (S + 1),), jnp.int32)])
    ssum, cnt, mn, mx = _sc_segment_stats(fv, off)
    bsum = (b1 + b2 + b3).reshape(1, DG)
    return _tc_combine(ssum, cnt, mn, mx, W1, W2, W3, bsum)


# speedup vs baseline: 1.4538x; 1.4538x over previous
import functools

import jax
import jax.numpy as jnp
from jax import lax
from jax.experimental import pallas as pl
from jax.experimental.pallas import tpu as pltpu
from jax.experimental.pallas import tpu_sc as plsc

N = 50000
DV = 256
DG = 256
S = 128

NC = 2
NS = 16
NW = NC * NS
SEG_PER_W = S // NW
LANES = 16
NJ = DV // LANES
GROUPS = 4
JPG = NJ // GROUPS
CH = 64
OFF_PAD = 144


def _sc_segment_stats(fv, offsets):
    mesh = plsc.VectorSubcoreMesh(core_axis_name="c", subcore_axis_name="s")
    out_type = tuple(
        jax.ShapeDtypeStruct((S * DV,), jnp.float32) for _ in range(4))

    @functools.partial(
        pl.kernel,
        mesh=mesh,
        out_type=out_type,
        scratch_types=[
            pltpu.VMEM((OFF_PAD,), jnp.int32),
            pltpu.VMEM((CH, DV), jnp.float32),
            pltpu.VMEM((CH, DV), jnp.float32),
            pltpu.VMEM((SEG_PER_W * DV,), jnp.float32),
            pltpu.VMEM((SEG_PER_W * DV,), jnp.float32),
            pltpu.VMEM((SEG_PER_W * DV,), jnp.float32),
            pltpu.VMEM((SEG_PER_W * DV,), jnp.float32),
            pltpu.SemaphoreType.DMA,
            pltpu.SemaphoreType.DMA,
        ],
    )
    def k(fv_hbm, off_hbm, sum_hbm, cnt_hbm, mn_hbm, mx_hbm,
          off_v, buf0, buf1, s_v, c_v, mn_v, mx_v, sem0, sem1):
        wid = lax.axis_index("s") * NC + lax.axis_index("c")
        pltpu.sync_copy(off_hbm, off_v)

        for kk in range(SEG_PER_W):
            seg = wid * SEG_PER_W + kk
            offv = off_v[pl.ds(seg, LANES)]
            a = offv[0]
            b = offv[1]
            n = b - a
            a8 = (a // 8) * 8
            nch = (b - a8 + CH - 1) // CH
            npair = (nch + 1) // 2

            def issue(c, buf, sem, a8=a8):
                row0 = jnp.minimum(a8 + c * CH, N - CH)
                pltpu.async_copy(fv_hbm.at[pl.ds(row0, CH)], buf, sem)

            def wait(buf, sem):
                pltpu.make_async_copy(
                    fv_hbm.at[pl.ds(0, CH)], buf, sem).wait()

            def process(buf, c, cy, a=a, b=b, a8=a8):
                row0 = a8 + c * CH
                w0 = jnp.minimum(row0, N - CH)
                lo = jnp.maximum(a, row0) - w0
                hi = jnp.maximum(jnp.minimum(b, row0 + CH) - w0, lo)
                cy = list(cy)
                for g in range(GROUPS):
                    sub = tuple(cy[3 * JPG * g: 3 * JPG * (g + 1)])

                    def row_body(r, sc, g=g, buf=buf):
                        out = []
                        for jj in range(JPG):
                            j = JPG * g + jj
                            v = buf[r, pl.ds(LANES * j, LANES)]
                            out += [
                                sc[3 * jj] + v,
                                jnp.minimum(sc[3 * jj + 1], v),
                                jnp.maximum(sc[3 * jj + 2], v),
                            ]
                        return tuple(out)

                    sub = plsc.parallel_loop(
                        lo, hi, 1, unroll=4, carry=sub)(row_body)
                    cy[3 * JPG * g: 3 * JPG * (g + 1)] = list(sub)
                return tuple(cy)

            carry = []
            for _ in range(NJ):
                carry += [
                    jnp.zeros((LANES,), jnp.float32),
                    jnp.full((LANES,), jnp.inf, jnp.float32),
                    jnp.full((LANES,), -jnp.inf, jnp.float32),
                ]
            carry = tuple(carry)

            @pl.when(nch > 0)
            def _():
                issue(0, buf0, sem0)

            def pair_body(p, cy):
                c0 = 2 * p
                wait(buf0, sem0)

                @pl.when(c0 + 1 < nch)
                def _():
                    issue(c0 + 1, buf1, sem1)

                cy = process(buf0, c0, cy)
                c1 = c0 + 1

                @pl.when(c1 < nch)
                def _():
                    wait(buf1, sem1)

                    @pl.when(c1 + 1 < nch)
                    def _():
                        issue(c1 + 1, buf0, sem0)

                cy = process(buf1, c1, cy)
                return cy

            carry = lax.fori_loop(0, npair, pair_body, carry)

            cntf = n.astype(jnp.float32)
            for j in range(NJ):
                ds = pl.ds(kk * DV + LANES * j, LANES)
                s_v[ds] = carry[3 * j]
                mn_v[ds] = carry[3 * j + 1]
                mx_v[ds] = carry[3 * j + 2]
                c_v[ds] = jnp.broadcast_to(cntf, (LANES,))

        wbase = wid * SEG_PER_W * DV
        wlen = SEG_PER_W * DV
        pltpu.sync_copy(s_v, sum_hbm.at[pl.ds(wbase, wlen)])
        pltpu.sync_copy(c_v, cnt_hbm.at[pl.ds(wbase, wlen)])
        pltpu.sync_copy(mn_v, mn_hbm.at[pl.ds(wbase, wlen)])
        pltpu.sync_copy(mx_v, mx_hbm.at[pl.ds(wbase, wlen)])

    outs = k(fv, offsets)
    return tuple(o.reshape(S, DV) for o in outs)


def _tc_combine(ssum, cnt, mn, mx, W1, W2, W3, bsum):
    def body(s_ref, c_ref, mn_ref, mx_ref, w1_ref, w2_ref, w3_ref, b_ref,
             o_ref):
        c = c_ref[...]
        mean = s_ref[...] / jnp.maximum(c, 1.0)
        ne = c > 0.0
        mnv = jnp.where(ne, mn_ref[...], 0.0)
        mxv = jnp.where(ne, mx_ref[...], 0.0)
        dn = (((1,), (1,)), ((), ()))
        acc = lax.dot_general(mean, w1_ref[...], dn,
                              precision=lax.Precision.HIGHEST,
                              preferred_element_type=jnp.float32)
        acc = acc + lax.dot_general(mnv, w2_ref[...], dn,
                                    precision=lax.Precision.HIGHEST,
                                    preferred_element_type=jnp.float32)
        acc = acc + lax.dot_general(mxv, w3_ref[...], dn,
                                    precision=lax.Precision.HIGHEST,
                                    preferred_element_type=jnp.float32)
        o_ref[...] = acc + b_ref[...]

    return pl.pallas_call(
        body,
        out_shape=jax.ShapeDtypeStruct((S, DG), jnp.float32),
    )(ssum, cnt, mn, mx, W1, W2, W3, bsum)


def kernel(fv, segment_ids, W1, b1, W2, b2, W3, b3):
    ids = segment_ids.astype(jnp.int32)
    off = jnp.searchsorted(ids, jnp.arange(S + 1, dtype=jnp.int32),
                           side="left", method="compare_all").astype(jnp.int32)
    off = jnp.concatenate([off, jnp.zeros((OFF_PAD - (S + 1),), jnp.int32)])
    ssum, cnt, mn, mx = _sc_segment_stats(fv, off)
    bsum = (b1 + b2 + b3).reshape(1, DG)
    return _tc_combine(ssum, cnt, mn, mx, W1, W2, W3, bsum)
